# baseline (speedup 1.0000x reference)
import functools

import jax
import jax.numpy as jnp
from jax import lax
from jax.experimental import pallas as pl
from jax.experimental.pallas import tpu as pltpu
from jax.experimental.pallas import tpu_sc as plsc

NC = 2
NS = 16
NW = NC * NS
L = 16
C = 128


def _build_calls(N, D, nch):
    NPAD = ((N + NS * C - 1) // (NS * C)) * (NS * C)
    npt = NPAD // NS
    nch2 = 2 * nch
    i32 = jnp.int32
    f32 = jnp.float32
    mesh = plsc.VectorSubcoreMesh(core_axis_name="c", subcore_axis_name="s")
    sc_params = pltpu.CompilerParams(needs_layout_passes=False)

    @functools.partial(
        pl.kernel,
        out_type=[jax.ShapeDtypeStruct((NW, nch, C), f32),
                  jax.ShapeDtypeStruct((NW, nch, C), f32)],
        mesh=mesh,
        scratch_types=[
            pltpu.VMEM((nch, C), i32),
            pltpu.VMEM((nch, C), i32),
            pltpu.VMEM((nch, C), f32),
            pltpu.VMEM((nch, C), f32),
            pltpu.VMEM((NPAD,), f32),
            pltpu.VMEM((NPAD,), f32),
            pltpu.VMEM_SHARED((NPAD,), f32),
            pltpu.VMEM_SHARED((NPAD,), f32),
        ],
        compiler_params=sc_params,
    )
    def prep(src_hbm, dst_hbm, w_hbm, a_s_hbm, a_t_hbm,
             sV, dV, wV, aV, degsV, degtV, degS, degT):
        cid = lax.axis_index("c")
        sid = lax.axis_index("s")
        wid = 2 * sid + cid

        def z16(i, _):
            degsV[pl.ds(i * L, L)] = jnp.zeros((L,), f32)
            return 0
        lax.fori_loop(0, NPAD // L, z16, 0)
        pltpu.sync_copy(degsV.at[pl.ds(sid * npt, npt)],
                        degS.at[pl.ds(sid * npt, npt)])
        pltpu.sync_copy(degsV.at[pl.ds(sid * npt, npt)],
                        degT.at[pl.ds(sid * npt, npt)])
        plsc.subcore_barrier()

        for slab_off in range(2):
            slab = 2 * sid + slab_off
            pltpu.sync_copy(src_hbm.at[slab], sV)
            pltpu.sync_copy(dst_hbm.at[slab], dV)
            pltpu.sync_copy(w_hbm.at[slab], wV)

            def srow(j, _):
                pltpu.sync_copy(wV.at[j], degS.at[sV.at[j]], add=True)
                pltpu.sync_copy(wV.at[j], degT.at[dV.at[j]], add=True)
                return 0
            lax.fori_loop(0, nch, srow, 0)
        plsc.subcore_barrier()

        pltpu.sync_copy(degS, degsV)
        pltpu.sync_copy(degT, degtV)

        def inv16(i, _):
            degsV[pl.ds(i * L, L)] = 1.0 / degsV[pl.ds(i * L, L)]
            degtV[pl.ds(i * L, L)] = 1.0 / degtV[pl.ds(i * L, L)]
            return 0
        lax.fori_loop(0, NPAD // L, inv16, 0)

        pltpu.sync_copy(src_hbm.at[wid], sV)
        pltpu.sync_copy(dst_hbm.at[wid], dV)
        pltpu.sync_copy(w_hbm.at[wid], wV)

        def arow_s(j, _):
            def agrp(k, _2):
                di = plsc.load_gather(degsV, [sV[j, pl.ds(k * L, L)]])
                aV[j, pl.ds(k * L, L)] = wV[j, pl.ds(k * L, L)] * di
                return 0
            lax.fori_loop(0, C // L, agrp, 0)
            return 0
        lax.fori_loop(0, nch, arow_s, 0)
        pltpu.sync_copy(aV, a_s_hbm.at[wid])

        def arow_t(j, _):
            def agrp(k, _2):
                di = plsc.load_gather(degtV, [dV[j, pl.ds(k * L, L)]])
                aV[j, pl.ds(k * L, L)] = wV[j, pl.ds(k * L, L)] * di
                return 0
            lax.fori_loop(0, C // L, agrp, 0)
            return 0
        lax.fori_loop(0, nch, arow_t, 0)
        pltpu.sync_copy(aV, a_t_hbm.at[wid])

    @functools.partial(
        pl.kernel,
        out_type=jax.ShapeDtypeStruct((NC, N, D), f32),
        mesh=mesh,
        scratch_types=[
            pltpu.VMEM((3, C), i32),
            pltpu.VMEM((3, C), i32),
            pltpu.VMEM((3, C), i32),
            pltpu.VMEM((3, C), i32),
            pltpu.VMEM((C, D), f32),
            pltpu.VMEM((C, D), f32),
            pltpu.VMEM_SHARED((NPAD, D), f32),
            pltpu.SemaphoreType.DMA,
            pltpu.SemaphoreType.DMA,
            pltpu.SemaphoreType.DMA,
            pltpu.SemaphoreType.DMA,
            pltpu.SemaphoreType.DMA,
            pltpu.SemaphoreType.DMA,
            pltpu.SemaphoreType.DMA,
            pltpu.SemaphoreType.DMA,
        ],
        compiler_params=sc_params,
    )
    def spmm(curr_hbm, edata_hbm, out_hbm, eV0, eV1, eV2, eV3,
             rows0, rows1, acc, se0, se1, se2, se3, sg0, sg1, ss0, ss1):
        cid = lax.axis_index("c")
        sid = lax.axis_index("s")
        eV = (eV0, eV1, eV2, eV3)
        rows = (rows0, rows1)
        se = (se0, se1, se2, se3)
        sg = (sg0, sg1)
        ss = (ss0, ss1)

        def zrow(i, _):
            def zc(k, _2):
                rows0[i, pl.ds(k * L, L)] = jnp.zeros((L,), f32)
                return 0
            lax.fori_loop(0, D // L, zc, 0)
            return 0
        lax.fori_loop(0, C, zrow, 0)
        for b in range(npt // C):
            pltpu.sync_copy(rows0, acc.at[pl.ds(sid * npt + b * C, C)])
        plsc.subcore_barrier()

        def stage(j, q):
            pltpu.async_copy(edata_hbm.at[cid, sid, j], eV[q], se[q])

        def wait_stage(j, q):
            pltpu.make_async_copy(
                edata_hbm.at[cid, sid, j], eV[q], se[q]).wait()

        def gather(q, b):
            pltpu.async_copy(curr_hbm.at[eV[q].at[0]], rows[b], sg[b])

        def wait_gather(q, b):
            pltpu.make_async_copy(
                curr_hbm.at[eV[q].at[0]], rows[b], sg[b]).wait()

        def scatter(q, b):
            pltpu.async_copy(rows[b], acc.at[eV[q].at[1]], ss[b], add=True)

        def wait_scatter(q, b):
            pltpu.make_async_copy(
                rows[b], acc.at[eV[q].at[1]], ss[b]).wait()

        def scale(q, b):
            def per_edge(e, _):
                bits = plsc.load_gather(
                    eV[q], [jnp.full((L,), 2, i32), jnp.full((L,), e, i32)])
                av = plsc.bitcast(bits, f32)
                for db in range(D // L):
                    rows[b][e, pl.ds(db * L, L)] = (
                        rows[b][e, pl.ds(db * L, L)] * av)
                return 0
            lax.fori_loop(0, C, per_edge, 0, unroll=8)

        stage(0, 0)
        stage(1, 1)
        stage(2, 2)
        wait_stage(0, 0)
        gather(0, 0)

        def body4(i, _):
            for u in range(4):
                j = 4 * i + u
                b = u % 2
                nb = 1 - b
                q = u
                qn = (u + 1) % 4
                qp = (u + 3) % 4

                @pl.when(j > 0)
                def _():
                    wait_scatter(qp, nb)

                @pl.when(j + 3 < nch2)
                def _():
                    stage(j + 3, qp)

                @pl.when(j + 1 < nch2)
                def _():
                    wait_stage(j + 1, qn)
                    gather(qn, nb)

                wait_gather(q, b)
                scale(q, b)
                scatter(q, b)
            return 0
        lax.fori_loop(0, nch2 // 4, body4, 0)
        wait_scatter((nch2 + 3) % 4, (nch2 + 1) % 2)
        plsc.subcore_barrier()
        full_tiles = N // npt
        rem = N % npt

        @pl.when(sid < full_tiles)
        def _():
            pltpu.sync_copy(acc.at[pl.ds(sid * npt, npt)],
                            out_hbm.at[cid, pl.ds(sid * npt, npt)])
        if rem:
            @pl.when(sid == full_tiles)
            def _():
                pltpu.sync_copy(
                    acc.at[pl.ds(full_tiles * npt, rem)],
                    out_hbm.at[cid, pl.ds(full_tiles * npt, rem)])

    BR = next(b for b in (400, 250, 200, 125, 100, 80, 50, 40, 25, 20, 16,
                          10, 8, 5, 4, 2, 1) if N % b == 0)

    def final_body(ws_ref, wt_ref, xs_ref, xt_ref, c1_ref, c2_ref, o_ref):
        o_ref[:, :D] = (ws_ref[0, 0] * xs_ref[...]
                        + ws_ref[1, 0] * c1_ref[0]
                        + ws_ref[2, 0] * c2_ref[0])
        o_ref[:, D:] = (wt_ref[0, 0] * xt_ref[...]
                        + wt_ref[1, 0] * c1_ref[1]
                        + wt_ref[2, 0] * c2_ref[1])

    final = pl.pallas_call(
        final_body,
        grid=(N // BR,),
        in_specs=[pl.BlockSpec(memory_space=pltpu.SMEM),
                  pl.BlockSpec(memory_space=pltpu.SMEM),
                  pl.BlockSpec((BR, D), lambda i: (i, 0)),
                  pl.BlockSpec((BR, D), lambda i: (i, 0)),
                  pl.BlockSpec((NC, BR, D), lambda i: (0, i, 0)),
                  pl.BlockSpec((NC, BR, D), lambda i: (0, i, 0))],
        out_specs=pl.BlockSpec((BR, 2 * D), lambda i: (i, 0)),
        out_shape=jax.ShapeDtypeStruct((N, 2 * D), f32),
    )

    return prep, spmm, final


def kernel(x_s, x_t, edge_index, edge_weight, w_s, w_t):
    N, D = x_s.shape
    E = edge_weight.shape[0]
    i32 = jnp.int32
    f32 = jnp.float32

    ET = E + N
    nch = (ET + NW * C - 1) // (NW * C)
    nch = (nch + 3) // 4 * 4
    EP = NW * nch * C
    pad = EP - ET
    loop_idx = jnp.arange(N, dtype=i32)
    pad_idx = jnp.arange(pad, dtype=i32) % N
    src_all = jnp.concatenate([edge_index[0], loop_idx, pad_idx])
    dst_all = jnp.concatenate([edge_index[1], loop_idx, pad_idx])
    w_all = jnp.concatenate(
        [edge_weight.astype(f32), jnp.full((N,), 0.5, f32),
         jnp.zeros((pad,), f32)])
    src_r = src_all.reshape(NW, nch, C)
    dst_r = dst_all.reshape(NW, nch, C)
    w_r = w_all.reshape(NW, nch, C)

    prep, spmm, final = _build_calls(N, D, nch)

    a_s, a_t = prep(src_r, dst_r, w_r)

    nch2 = 2 * nch

    def _ed(g, s, a):
        return jnp.stack(
            [g.reshape(NS, nch2, C), s.reshape(NS, nch2, C),
             jax.lax.bitcast_convert_type(a, i32).reshape(NS, nch2, C)],
            axis=2)

    edata = jnp.stack(
        [_ed(src_all, dst_all, a_s.reshape(EP)),
         _ed(dst_all + N, src_all, a_t.reshape(EP))], axis=0)

    x2 = jnp.concatenate([x_s, x_t], axis=0)
    c1 = spmm(x2, edata)
    c2 = spmm(c1.reshape(2 * N, D), edata)

    return final(w_s, w_t, x_s, x_t, c1, c2)


# speedup vs baseline: 1.0016x; 1.0016x over previous
import functools

import jax
import jax.numpy as jnp
from jax import lax
from jax.experimental import pallas as pl
from jax.experimental.pallas import tpu as pltpu
from jax.experimental.pallas import tpu_sc as plsc

NC = 2
NS = 16
NW = NC * NS
L = 16
C = 128


def _build_calls(N, D, nch):
    NPAD = ((N + NS * C - 1) // (NS * C)) * (NS * C)
    npt = NPAD // NS
    nch2 = 2 * nch
    i32 = jnp.int32
    f32 = jnp.float32
    mesh = plsc.VectorSubcoreMesh(core_axis_name="c", subcore_axis_name="s")
    sc_params = pltpu.CompilerParams(needs_layout_passes=False)

    @functools.partial(
        pl.kernel,
        out_type=[jax.ShapeDtypeStruct((NW, nch, C), f32),
                  jax.ShapeDtypeStruct((NW, nch, C), f32)],
        mesh=mesh,
        scratch_types=[
            pltpu.VMEM((nch, C), i32),
            pltpu.VMEM((nch, C), i32),
            pltpu.VMEM((nch, C), f32),
            pltpu.VMEM((nch, C), f32),
            pltpu.VMEM((NPAD,), f32),
            pltpu.VMEM((NPAD,), f32),
            pltpu.VMEM_SHARED((NPAD,), f32),
            pltpu.VMEM_SHARED((NPAD,), f32),
        ],
        compiler_params=sc_params,
    )
    def prep(src_hbm, dst_hbm, w_hbm, a_s_hbm, a_t_hbm,
             sV, dV, wV, aV, degsV, degtV, degS, degT):
        cid = lax.axis_index("c")
        sid = lax.axis_index("s")
        wid = 2 * sid + cid

        def z16(i, _):
            degsV[pl.ds(i * L, L)] = jnp.zeros((L,), f32)
            return 0
        lax.fori_loop(0, NPAD // L, z16, 0)
        pltpu.sync_copy(degsV.at[pl.ds(sid * npt, npt)],
                        degS.at[pl.ds(sid * npt, npt)])
        pltpu.sync_copy(degsV.at[pl.ds(sid * npt, npt)],
                        degT.at[pl.ds(sid * npt, npt)])
        plsc.subcore_barrier()

        for slab_off in range(2):
            slab = 2 * sid + slab_off
            pltpu.sync_copy(src_hbm.at[slab], sV)
            pltpu.sync_copy(dst_hbm.at[slab], dV)
            pltpu.sync_copy(w_hbm.at[slab], wV)

            def srow(j, _):
                pltpu.sync_copy(wV.at[j], degS.at[sV.at[j]], add=True)
                pltpu.sync_copy(wV.at[j], degT.at[dV.at[j]], add=True)
                return 0
            lax.fori_loop(0, nch, srow, 0)
        plsc.subcore_barrier()

        pltpu.sync_copy(degS, degsV)
        pltpu.sync_copy(degT, degtV)

        def inv16(i, _):
            degsV[pl.ds(i * L, L)] = 1.0 / degsV[pl.ds(i * L, L)]
            degtV[pl.ds(i * L, L)] = 1.0 / degtV[pl.ds(i * L, L)]
            return 0
        lax.fori_loop(0, NPAD // L, inv16, 0)

        pltpu.sync_copy(src_hbm.at[wid], sV)
        pltpu.sync_copy(dst_hbm.at[wid], dV)
        pltpu.sync_copy(w_hbm.at[wid], wV)

        def arow_s(j, _):
            def agrp(k, _2):
                di = plsc.load_gather(degsV, [sV[j, pl.ds(k * L, L)]])
                aV[j, pl.ds(k * L, L)] = wV[j, pl.ds(k * L, L)] * di
                return 0
            lax.fori_loop(0, C // L, agrp, 0)
            return 0
        lax.fori_loop(0, nch, arow_s, 0)
        pltpu.sync_copy(aV, a_s_hbm.at[wid])

        def arow_t(j, _):
            def agrp(k, _2):
                di = plsc.load_gather(degtV, [dV[j, pl.ds(k * L, L)]])
                aV[j, pl.ds(k * L, L)] = wV[j, pl.ds(k * L, L)] * di
                return 0
            lax.fori_loop(0, C // L, agrp, 0)
            return 0
        lax.fori_loop(0, nch, arow_t, 0)
        pltpu.sync_copy(aV, a_t_hbm.at[wid])

    @functools.partial(
        pl.kernel,
        out_type=jax.ShapeDtypeStruct((NC, N, D), f32),
        mesh=mesh,
        scratch_types=[
            pltpu.VMEM((3, C), i32),
            pltpu.VMEM((3, C), i32),
            pltpu.VMEM((3, C), i32),
            pltpu.VMEM((3, C), i32),
            pltpu.VMEM((C, D), f32),
            pltpu.VMEM((C, D), f32),
            pltpu.VMEM_SHARED((NPAD, D), f32),
            pltpu.SemaphoreType.DMA,
            pltpu.SemaphoreType.DMA,
            pltpu.SemaphoreType.DMA,
            pltpu.SemaphoreType.DMA,
            pltpu.SemaphoreType.DMA,
            pltpu.SemaphoreType.DMA,
            pltpu.SemaphoreType.DMA,
            pltpu.SemaphoreType.DMA,
        ],
        compiler_params=sc_params,
    )
    def spmm(curr_hbm, edata_hbm, out_hbm, eV0, eV1, eV2, eV3,
             rows0, rows1, acc, se0, se1, se2, se3, sg0, sg1, ss0, ss1):
        cid = lax.axis_index("c")
        sid = lax.axis_index("s")
        eV = (eV0, eV1, eV2, eV3)
        rows = (rows0, rows1)
        se = (se0, se1, se2, se3)
        sg = (sg0, sg1)
        ss = (ss0, ss1)

        def zrow(i, _):
            def zc(k, _2):
                rows0[i, pl.ds(k * L, L)] = jnp.zeros((L,), f32)
                return 0
            lax.fori_loop(0, D // L, zc, 0)
            return 0
        lax.fori_loop(0, C, zrow, 0)
        for b in range(npt // C):
            pltpu.sync_copy(rows0, acc.at[pl.ds(sid * npt + b * C, C)])
        plsc.subcore_barrier()

        def stage(j, q):
            pltpu.async_copy(edata_hbm.at[cid, sid, j], eV[q], se[q])

        def wait_stage(j, q):
            pltpu.make_async_copy(
                edata_hbm.at[cid, sid, j], eV[q], se[q]).wait()

        def gather(q, b):
            pltpu.async_copy(curr_hbm.at[eV[q].at[0]], rows[b], sg[b])

        def wait_gather(q, b):
            pltpu.make_async_copy(
                curr_hbm.at[eV[q].at[0]], rows[b], sg[b]).wait()

        def scatter(q, b):
            pltpu.async_copy(rows[b], acc.at[eV[q].at[1]], ss[b], add=True)

        def wait_scatter(q, b):
            pltpu.make_async_copy(
                rows[b], acc.at[eV[q].at[1]], ss[b]).wait()

        def scale(q, b):
            def per_edge(e, _):
                bits = plsc.load_gather(
                    eV[q], [jnp.full((L,), 2, i32), jnp.full((L,), e, i32)])
                av = plsc.bitcast(bits, f32)
                for db in range(D // L):
                    rows[b][e, pl.ds(db * L, L)] = (
                        rows[b][e, pl.ds(db * L, L)] * av)
                return 0
            lax.fori_loop(0, C, per_edge, 0, unroll=4)

        stage(0, 0)
        stage(1, 1)
        stage(2, 2)
        wait_stage(0, 0)
        gather(0, 0)

        def body4(i, _):
            for u in range(4):
                j = 4 * i + u
                b = u % 2
                nb = 1 - b
                q = u
                qn = (u + 1) % 4
                qp = (u + 3) % 4

                @pl.when(j > 0)
                def _():
                    wait_scatter(qp, nb)

                @pl.when(j + 3 < nch2)
                def _():
                    stage(j + 3, qp)

                @pl.when(j + 1 < nch2)
                def _():
                    wait_stage(j + 1, qn)
                    gather(qn, nb)

                wait_gather(q, b)
                scale(q, b)
                scatter(q, b)
            return 0
        lax.fori_loop(0, nch2 // 4, body4, 0)
        wait_scatter((nch2 + 3) % 4, (nch2 + 1) % 2)
        plsc.subcore_barrier()
        full_tiles = N // npt
        rem = N % npt

        @pl.when(sid < full_tiles)
        def _():
            pltpu.sync_copy(acc.at[pl.ds(sid * npt, npt)],
                            out_hbm.at[cid, pl.ds(sid * npt, npt)])
        if rem:
            @pl.when(sid == full_tiles)
            def _():
                pltpu.sync_copy(
                    acc.at[pl.ds(full_tiles * npt, rem)],
                    out_hbm.at[cid, pl.ds(full_tiles * npt, rem)])

    BR = next(b for b in (400, 250, 200, 125, 100, 80, 50, 40, 25, 20, 16,
                          10, 8, 5, 4, 2, 1) if N % b == 0)

    def final_body(ws_ref, wt_ref, xs_ref, xt_ref, c1_ref, c2_ref, o_ref):
        o_ref[:, :D] = (ws_ref[0, 0] * xs_ref[...]
                        + ws_ref[1, 0] * c1_ref[0]
                        + ws_ref[2, 0] * c2_ref[0])
        o_ref[:, D:] = (wt_ref[0, 0] * xt_ref[...]
                        + wt_ref[1, 0] * c1_ref[1]
                        + wt_ref[2, 0] * c2_ref[1])

    final = pl.pallas_call(
        final_body,
        grid=(N // BR,),
        in_specs=[pl.BlockSpec(memory_space=pltpu.SMEM),
                  pl.BlockSpec(memory_space=pltpu.SMEM),
                  pl.BlockSpec((BR, D), lambda i: (i, 0)),
                  pl.BlockSpec((BR, D), lambda i: (i, 0)),
                  pl.BlockSpec((NC, BR, D), lambda i: (0, i, 0)),
                  pl.BlockSpec((NC, BR, D), lambda i: (0, i, 0))],
        out_specs=pl.BlockSpec((BR, 2 * D), lambda i: (i, 0)),
        out_shape=jax.ShapeDtypeStruct((N, 2 * D), f32),
    )

    return prep, spmm, final


def kernel(x_s, x_t, edge_index, edge_weight, w_s, w_t):
    N, D = x_s.shape
    E = edge_weight.shape[0]
    i32 = jnp.int32
    f32 = jnp.float32

    ET = E + N
    nch = (ET + NW * C - 1) // (NW * C)
    nch = (nch + 3) // 4 * 4
    EP = NW * nch * C
    pad = EP - ET
    loop_idx = jnp.arange(N, dtype=i32)
    pad_idx = jnp.arange(pad, dtype=i32) % N
    src_all = jnp.concatenate([edge_index[0], loop_idx, pad_idx])
    dst_all = jnp.concatenate([edge_index[1], loop_idx, pad_idx])
    w_all = jnp.concatenate(
        [edge_weight.astype(f32), jnp.full((N,), 0.5, f32),
         jnp.zeros((pad,), f32)])
    src_r = src_all.reshape(NW, nch, C)
    dst_r = dst_all.reshape(NW, nch, C)
    w_r = w_all.reshape(NW, nch, C)

    prep, spmm, final = _build_calls(N, D, nch)

    a_s, a_t = prep(src_r, dst_r, w_r)

    nch2 = 2 * nch

    def _ed(g, s, a):
        return jnp.stack(
            [g.reshape(NS, nch2, C), s.reshape(NS, nch2, C),
             jax.lax.bitcast_convert_type(a, i32).reshape(NS, nch2, C)],
            axis=2)

    edata = jnp.stack(
        [_ed(src_all, dst_all, a_s.reshape(EP)),
         _ed(dst_all + N, src_all, a_t.reshape(EP))], axis=0)

    x2 = jnp.concatenate([x_s, x_t], axis=0)
    c1 = spmm(x2, edata)
    c2 = spmm(c1.reshape(2 * N, D), edata)

    return final(w_s, w_t, x_s, x_t, c1, c2)
